# Initial kernel scaffold; baseline (speedup 1.0000x reference)
import jax
import jax.numpy as jnp
from jax.experimental import pallas as pl


def kernel(input_ids, emb_weight):
    raise NotImplementedError("write your pallas kernel here")


# speedup vs baseline: 2.6410x; 2.6410x over previous
import functools

import jax
import jax.numpy as jnp
from jax import lax
from jax.experimental import pallas as pl
from jax.experimental.pallas import tpu as pltpu
from jax.experimental.pallas import tpu_sc as plsc

VOCAB = 100000
HIDDEN = 128
BATCH = 4096 * 50
NUM_CORES = 2
NUM_SUBCORES = 16
NW = NUM_CORES * NUM_SUBCORES
BPW = BATCH // NW
CHUNK = 128
NCHUNK = BPW // CHUNK

_mesh = plsc.VectorSubcoreMesh(core_axis_name="c", subcore_axis_name="s")


@functools.partial(
    pl.kernel,
    mesh=_mesh,
    out_type=jax.ShapeDtypeStruct((NW, NCHUNK, CHUNK, HIDDEN), jnp.float32),
    scratch_types=[
        pltpu.VMEM((NCHUNK, CHUNK), jnp.int32),
        pltpu.VMEM((CHUNK, HIDDEN), jnp.float32),
        pltpu.SemaphoreType.DMA,
    ],
)
def _emb_gather(ids_hbm, table_hbm, out_hbm, idx_v, rows_v, sem):
    wid = lax.axis_index("s") * NUM_CORES + lax.axis_index("c")
    pltpu.sync_copy(ids_hbm.at[wid], idx_v)

    def body(j, carry):
        pltpu.async_copy(table_hbm.at[idx_v.at[j]], rows_v, sem).wait()
        pltpu.sync_copy(rows_v, out_hbm.at[wid, j])
        return carry

    lax.fori_loop(0, NCHUNK, body, 0)


def kernel(input_ids, emb_weight):
    ids = input_ids.reshape(NW, NCHUNK, CHUNK).astype(jnp.int32)
    out = _emb_gather(ids, emb_weight)
    h = out.reshape(4096, 50, HIDDEN)
    return (h, h)


# speedup vs baseline: 2.9111x; 1.1023x over previous
import functools

import jax
import jax.numpy as jnp
from jax import lax
from jax.experimental import pallas as pl
from jax.experimental.pallas import tpu as pltpu
from jax.experimental.pallas import tpu_sc as plsc

VOCAB = 100000
HIDDEN = 128
BATCH = 4096 * 50
NUM_CORES = 2
NUM_SUBCORES = 16
NW = NUM_CORES * NUM_SUBCORES
BPW = BATCH // NW
CHUNK = 128
NCHUNK = BPW // CHUNK
NBUF = 5
PREF = 3
NGROUPS = NCHUNK // NBUF

_mesh = plsc.VectorSubcoreMesh(core_axis_name="c", subcore_axis_name="s")


@functools.partial(
    pl.kernel,
    mesh=_mesh,
    out_type=jax.ShapeDtypeStruct((NW, NCHUNK, CHUNK, HIDDEN), jnp.float32),
    scratch_types=[pltpu.VMEM((NCHUNK, CHUNK), jnp.int32)]
    + [pltpu.VMEM((CHUNK, HIDDEN), jnp.float32) for _ in range(NBUF)]
    + [pltpu.SemaphoreType.DMA for _ in range(2 * NBUF)],
)
def _emb_gather(ids_hbm, table_hbm, out_hbm, idx_v, *bufs):
    rows = bufs[:NBUF]
    gsem = bufs[NBUF:2 * NBUF]
    ssem = bufs[2 * NBUF:]
    wid = lax.axis_index("s") * NUM_CORES + lax.axis_index("c")
    pltpu.sync_copy(ids_hbm.at[wid], idx_v)

    def gather(j, b):
        return pltpu.make_async_copy(table_hbm.at[idx_v.at[j]], rows[b], gsem[b])

    def store(j, b):
        return pltpu.make_async_copy(rows[b], out_hbm.at[wid, j], ssem[b])

    for j in range(PREF):
        gather(j, j).start()

    def step(j, b, first_round):
        bb = (b + PREF) % NBUF
        if not first_round:
            store(j, bb).wait()
        gather(j + PREF, bb).start()
        gather(j, b).wait()
        store(j, b).start()

    for b in range(NBUF):
        step(b, b, first_round=(b + PREF < NBUF))

    def group(g, carry):
        for b in range(NBUF):
            step(g * NBUF + b, b, first_round=False)
        return carry

    lax.fori_loop(1, NGROUPS - 1, group, 0)

    for b in range(NBUF):
        j = (NGROUPS - 1) * NBUF + b
        if j + PREF < NCHUNK:
            bb = (b + PREF) % NBUF
            store(j, bb).wait()
            gather(j + PREF, bb).start()
        gather(j, b).wait()
        store(j, b).start()
    for b in range(NBUF):
        store(0, b).wait()


def kernel(input_ids, emb_weight):
    ids = input_ids.reshape(NW, NCHUNK, CHUNK).astype(jnp.int32)
    out = _emb_gather(ids, emb_weight)
    h = out.reshape(4096, 50, HIDDEN)
    return (h, h)
